# baseline (speedup 1.0000x reference)
import functools

import jax
import jax.numpy as jnp
from jax import lax
from jax.experimental import pallas as pl
from jax.experimental.pallas import tpu as pltpu
from jax.experimental.pallas import tpu_sc as plsc

N = 8192
NUM_EVENTS = 8
K = 40
EPS = 1e-5

RB = 256
CT = 2048
NT = N // CT
NRB = N // RB
INF = 3.0e38
NEG = -3.0e38


def _conv_body(lims_ref, rows_ref, cols_ref, out_ref, d2_ref):
    i = pl.program_id(0)
    t_lo = lims_ref[i, 0]
    t_hi = lims_ref[i, 1]

    s_r = [rows_ref[:, f:f + 1] for f in range(4)]
    sq_r = rows_ref[:, 4:5]
    b_r = rows_ref[:, 5:6]

    def bf(v):
        return v.astype(jnp.bfloat16).astype(jnp.float32)

    sb_r = [bf(v) for v in s_r]

    lane = jax.lax.broadcasted_iota(jnp.int32, (RB, CT), 1)
    lane_f = lane.astype(jnp.float32)

    def mat_body(j, carry):
        c = cols_ref[j]
        dot = (sb_r[0] * bf(c[0:1]) + sb_r[1] * bf(c[1:2])
               + sb_r[2] * bf(c[2:3]) + sb_r[3] * bf(c[3:4]))
        d2 = sq_r + c[4:5] - 2.0 * dot
        pad = jnp.float32(1e9) + (lane_f + jnp.float32(CT)
                                  * j.astype(jnp.float32)) * 64.0
        d2 = jnp.where(b_r != c[5:6], pad, d2)
        d2_ref[j] = d2
        return carry

    jax.lax.fori_loop(t_lo, t_hi, mat_body, 0)

    BIGI = jnp.int32(2 ** 30)

    t_prev = jnp.full((RB, 1), NEG, jnp.float32)
    j_prev = jnp.full((RB, 1), -1, jnp.int32)
    for k in range(K):
        def sweep(jt, mj):
            m, jj = mj
            d2 = d2_ref[jt]
            jl = j_prev - jt * CT
            elig = (d2 > t_prev) | ((d2 == t_prev) & (lane > jl))
            d2e = jnp.where(elig, d2, INF)
            m_t = jnp.min(d2e, axis=1, keepdims=True)
            j_t = jnp.min(jnp.where(d2e == m_t, lane, BIGI), axis=1,
                          keepdims=True) + jt * CT
            take = (m_t < m) | ((m_t == m) & (j_t < jj))
            return (jnp.where(take, m_t, m), jnp.where(take, j_t, jj))

        m0 = jnp.full((RB, 1), INF, jnp.float32)
        j0 = jnp.full((RB, 1), BIGI, jnp.int32)
        t_prev, j_prev = jax.lax.fori_loop(t_lo, t_hi, sweep, (m0, j0))
        out_ref[:, k:k + 1] = j_prev

def _conv_topk(rows, cols3, lims):
    return pl.pallas_call(
        _conv_body,
        grid=(NRB,),
        in_specs=[
            pl.BlockSpec(memory_space=pltpu.SMEM),
            pl.BlockSpec((RB, 8), lambda i: (i, 0)),
            pl.BlockSpec((NT, 8, CT), lambda i: (0, 0, 0)),
        ],
        out_specs=pl.BlockSpec((RB, K), lambda i: (i, 0)),
        out_shape=jax.ShapeDtypeStruct((N, K), jnp.int32),
        scratch_shapes=[pltpu.VMEM((NT, RB, CT), jnp.float32)],
    )(lims, rows, cols3)


GB = N * K
NW = 32
B_PER_W = GB // NW
GCH = 128
NCH = B_PER_W // GCH


def _sc_gather_fn(tab_hbm, idx_hbm, out_hbm, idx_v, buf0, buf1, sem0, sem1):
    wid = lax.axis_index("s") * 2 + lax.axis_index("c")
    base = wid * B_PER_W
    pltpu.sync_copy(idx_hbm.at[pl.ds(base, B_PER_W)], idx_v)

    def step(ch, carry):
        cbase = ch * (2 * GCH)
        pltpu.async_copy(tab_hbm.at[idx_v.at[pl.ds(cbase, GCH)]],
                         buf0, sem0)
        pltpu.async_copy(tab_hbm.at[idx_v.at[pl.ds(cbase + GCH, GCH)]],
                         buf1, sem1)
        pltpu.make_async_copy(tab_hbm.at[idx_v.at[pl.ds(cbase, GCH)]],
                              buf0, sem0).wait()
        pltpu.sync_copy(buf0, out_hbm.at[pl.ds(base + cbase, GCH)])
        pltpu.make_async_copy(tab_hbm.at[idx_v.at[pl.ds(cbase + GCH, GCH)]],
                              buf1, sem1).wait()
        pltpu.sync_copy(buf1, out_hbm.at[pl.ds(base + cbase + GCH, GCH)])
        return carry

    jax.lax.fori_loop(0, NCH // 2, step, 0)


_sc_gather = functools.partial(
    pl.kernel, mesh=plsc.VectorSubcoreMesh(core_axis_name="c",
                                           subcore_axis_name="s"),
    out_type=jax.ShapeDtypeStruct((GB, 128), jnp.float32),
    scratch_types=[
        pltpu.VMEM((B_PER_W,), jnp.int32),
        pltpu.VMEM((GCH, 128), jnp.float32),
        pltpu.VMEM((GCH, 128), jnp.float32),
        pltpu.SemaphoreType.DMA,
        pltpu.SemaphoreType.DMA,
    ],
)(_sc_gather_fn)


def _linear(x, p):
    y = x @ p["W"]
    if "b" in p:
        y = y + p["b"]
    return y


def _batchnorm(x, p):
    mu = jnp.mean(x, axis=0)
    var = jnp.var(x, axis=0)
    return (x - mu) / jnp.sqrt(var + EPS) * p["gamma"] + p["beta"]


def _global_exchange(x, batch, num_segments):
    ones = jnp.ones((x.shape[0],), jnp.float32)
    counts = jax.ops.segment_sum(ones, batch, num_segments=num_segments)
    mean = jax.ops.segment_sum(
        x, batch, num_segments=num_segments) / jnp.maximum(counts, 1.0)[:, None]
    mn = jax.ops.segment_min(x, batch, num_segments=num_segments)
    mx = jax.ops.segment_max(x, batch, num_segments=num_segments)
    stats = jnp.concatenate([mean, mn, mx], axis=-1)
    return jnp.concatenate([stats[batch], x], axis=-1)


def _gravnet_conv(x, batch, batchf, lims, p):
    s = _linear(x, p["lin_s"])
    h = _linear(x, p["lin_h"])
    sq = jnp.sum(s * s, axis=-1)

    rows = jnp.concatenate(
        [s, sq[:, None], batchf[:, None], jnp.zeros((N, 2), jnp.float32)],
        axis=1)
    cols = jnp.concatenate(
        [s.T, sq[None, :], batchf[None, :], jnp.zeros((2, N), jnp.float32)],
        axis=0)
    cols3 = cols.reshape(8, NT, CT).transpose(1, 0, 2)

    idx = _conv_topk(rows, cols3, lims)
    tab = jnp.concatenate([s, h, jnp.zeros((N, 102), jnp.float32)], axis=1)
    rows_g = _sc_gather(tab, idx.reshape(GB)).reshape(N, K, 128)
    sj = rows_g[:, :, 0:4]
    hj = rows_g[:, :, 4:26]
    diff = sj - s[:, None, :]
    w = jnp.exp(-10.0 * jnp.sum(diff * diff, axis=-1))
    msg = hj * w[:, :, None]
    agg = jnp.concatenate([jnp.mean(msg, axis=1), jnp.max(msg, axis=1)],
                          axis=-1)
    return x @ p["lin_out1"]["W"] + _linear(agg, p["lin_out2"])


def _gravnet_block(x, batch, batchf, lims, p, ns):
    x = _gravnet_conv(x, batch, batchf, lims, p["conv"])
    x = _batchnorm(x, p["bn1"])
    x = jnp.tanh(_linear(x, p["lin1"]))
    x = _batchnorm(x, p["bn2"])
    x = jnp.tanh(_linear(x, p["lin2"]))
    x = _global_exchange(x, batch, ns)
    x = jnp.tanh(_linear(x, p["out_lin"]))
    x = _batchnorm(x, p["out_bn"])
    return x


def kernel(x, batch, params):
    batchf = batch.astype(jnp.float32)
    starts = jnp.searchsorted(batch, jnp.arange(NUM_EVENTS, dtype=jnp.int32),
                              side="left").astype(jnp.int32)
    ends = jnp.searchsorted(batch, jnp.arange(NUM_EVENTS, dtype=jnp.int32),
                            side="right").astype(jnp.int32)
    b_first = batch[::RB]
    b_last = batch[RB - 1::RB]
    lo = (starts[b_first] // CT).astype(jnp.int32)
    hi = ((ends[b_last] + CT - 1) // CT).astype(jnp.int32)
    lims = jnp.stack([lo, hi], axis=1)

    x = _batchnorm(x, params["bn0"])
    x = _global_exchange(x, batch, NUM_EVENTS)
    x = _linear(x, params["input"])
    outs = []
    for bp in params["blocks"]:
        x = _gravnet_block(x, batch, batchf, lims, bp, NUM_EVENTS)
        outs.append(x)
    x = jnp.concatenate(outs, axis=-1)
    for dp in params["dense"]:
        x = _batchnorm(jax.nn.relu(_linear(x, dp["lin"])), dp["bn"])
    x = jax.nn.relu(_linear(x, params["out1"]))
    x = jax.nn.relu(_linear(x, params["out2"]))
    return _linear(x, params["out3"])


# speedup vs baseline: 1.2494x; 1.2494x over previous
import functools

import jax
import jax.numpy as jnp
from jax import lax
from jax.experimental import pallas as pl
from jax.experimental.pallas import tpu as pltpu
from jax.experimental.pallas import tpu_sc as plsc

N = 8192
NUM_EVENTS = 8
K = 40
EPS = 1e-5

RB = 256
CT = 1024
NT = N // CT
NRB = N // RB
INF = 3.0e38
NEG = -3.0e38


def _conv_body(lims_ref, rows_ref, cols_ref, out_ref, d2_ref):
    i = pl.program_id(0)
    t_lo = lims_ref[i, 0]
    t_hi = lims_ref[i, 1]

    s_r = [rows_ref[:, f:f + 1] for f in range(4)]
    sq_r = rows_ref[:, 4:5]
    b_r = rows_ref[:, 5:6]

    def bf(v):
        return v.astype(jnp.bfloat16).astype(jnp.float32)

    sb_r = [bf(v) for v in s_r]

    lane = jax.lax.broadcasted_iota(jnp.int32, (RB, CT), 1)
    lane_f = lane.astype(jnp.float32)

    def mat_body(j, carry):
        c = cols_ref[j]
        dot = (sb_r[0] * bf(c[0:1]) + sb_r[1] * bf(c[1:2])
               + sb_r[2] * bf(c[2:3]) + sb_r[3] * bf(c[3:4]))
        d2 = sq_r + c[4:5] - 2.0 * dot
        pad = jnp.float32(1e9) + (lane_f + jnp.float32(CT)
                                  * j.astype(jnp.float32)) * 64.0
        d2 = jnp.where(b_r != c[5:6], pad, d2)
        d2_ref[j] = d2
        return carry

    jax.lax.fori_loop(t_lo, t_hi, mat_body, 0)

    BIGI = jnp.int32(2 ** 30)

    t_prev = jnp.full((RB, 1), NEG, jnp.float32)
    j_prev = jnp.full((RB, 1), -1, jnp.int32)
    for k in range(K):
        def sweep(jt, mj):
            m, jj = mj
            d2 = d2_ref[jt]
            jl = j_prev - jt * CT
            elig = (d2 > t_prev) | ((d2 == t_prev) & (lane > jl))
            d2e = jnp.where(elig, d2, INF)
            m_t = jnp.min(d2e, axis=1, keepdims=True)
            j_t = jnp.min(jnp.where(d2e == m_t, lane, BIGI), axis=1,
                          keepdims=True) + jt * CT
            take = (m_t < m) | ((m_t == m) & (j_t < jj))
            return (jnp.where(take, m_t, m), jnp.where(take, j_t, jj))

        m0 = jnp.full((RB, 1), INF, jnp.float32)
        j0 = jnp.full((RB, 1), BIGI, jnp.int32)
        t_prev, j_prev = jax.lax.fori_loop(t_lo, t_hi, sweep, (m0, j0))
        out_ref[:, k:k + 1] = j_prev

def _conv_topk(rows, cols3, lims):
    return pl.pallas_call(
        _conv_body,
        grid=(NRB,),
        in_specs=[
            pl.BlockSpec(memory_space=pltpu.SMEM),
            pl.BlockSpec((RB, 8), lambda i: (i, 0)),
            pl.BlockSpec((NT, 8, CT), lambda i: (0, 0, 0)),
        ],
        out_specs=pl.BlockSpec((RB, K), lambda i: (i, 0)),
        out_shape=jax.ShapeDtypeStruct((N, K), jnp.int32),
        scratch_shapes=[pltpu.VMEM((NT, RB, CT), jnp.float32)],
    )(lims, rows, cols3)


GB = N * K
NW = 32
B_PER_W = GB // NW
GCH = 128
NCH = B_PER_W // GCH


def _sc_gather_fn(tab_hbm, idx_hbm, out_hbm, idx_v, buf0, buf1, sem0, sem1):
    wid = lax.axis_index("s") * 2 + lax.axis_index("c")
    base = wid * B_PER_W
    pltpu.sync_copy(idx_hbm.at[pl.ds(base, B_PER_W)], idx_v)

    def step(ch, carry):
        cbase = ch * (2 * GCH)
        pltpu.async_copy(tab_hbm.at[idx_v.at[pl.ds(cbase, GCH)]],
                         buf0, sem0)
        pltpu.async_copy(tab_hbm.at[idx_v.at[pl.ds(cbase + GCH, GCH)]],
                         buf1, sem1)
        pltpu.make_async_copy(tab_hbm.at[idx_v.at[pl.ds(cbase, GCH)]],
                              buf0, sem0).wait()
        pltpu.sync_copy(buf0, out_hbm.at[pl.ds(base + cbase, GCH)])
        pltpu.make_async_copy(tab_hbm.at[idx_v.at[pl.ds(cbase + GCH, GCH)]],
                              buf1, sem1).wait()
        pltpu.sync_copy(buf1, out_hbm.at[pl.ds(base + cbase + GCH, GCH)])
        return carry

    jax.lax.fori_loop(0, NCH // 2, step, 0)


_sc_gather = functools.partial(
    pl.kernel, mesh=plsc.VectorSubcoreMesh(core_axis_name="c",
                                           subcore_axis_name="s"),
    out_type=jax.ShapeDtypeStruct((GB, 128), jnp.float32),
    scratch_types=[
        pltpu.VMEM((B_PER_W,), jnp.int32),
        pltpu.VMEM((GCH, 128), jnp.float32),
        pltpu.VMEM((GCH, 128), jnp.float32),
        pltpu.SemaphoreType.DMA,
        pltpu.SemaphoreType.DMA,
    ],
)(_sc_gather_fn)


def _linear(x, p):
    y = x @ p["W"]
    if "b" in p:
        y = y + p["b"]
    return y


def _batchnorm(x, p):
    mu = jnp.mean(x, axis=0)
    var = jnp.var(x, axis=0)
    return (x - mu) / jnp.sqrt(var + EPS) * p["gamma"] + p["beta"]


def _global_exchange(x, batch, num_segments):
    ones = jnp.ones((x.shape[0],), jnp.float32)
    counts = jax.ops.segment_sum(ones, batch, num_segments=num_segments)
    mean = jax.ops.segment_sum(
        x, batch, num_segments=num_segments) / jnp.maximum(counts, 1.0)[:, None]
    mn = jax.ops.segment_min(x, batch, num_segments=num_segments)
    mx = jax.ops.segment_max(x, batch, num_segments=num_segments)
    stats = jnp.concatenate([mean, mn, mx], axis=-1)
    return jnp.concatenate([stats[batch], x], axis=-1)


def _gravnet_conv(x, batch, batchf, lims, p):
    s = _linear(x, p["lin_s"])
    h = _linear(x, p["lin_h"])
    sq = jnp.sum(s * s, axis=-1)

    rows = jnp.concatenate(
        [s, sq[:, None], batchf[:, None], jnp.zeros((N, 2), jnp.float32)],
        axis=1)
    cols = jnp.concatenate(
        [s.T, sq[None, :], batchf[None, :], jnp.zeros((2, N), jnp.float32)],
        axis=0)
    cols3 = cols.reshape(8, NT, CT).transpose(1, 0, 2)

    idx = _conv_topk(rows, cols3, lims)
    tab = jnp.concatenate([s, h, jnp.zeros((N, 102), jnp.float32)], axis=1)
    rows_g = _sc_gather(tab, idx.reshape(GB)).reshape(N, K, 128)
    sj = rows_g[:, :, 0:4]
    hj = rows_g[:, :, 4:26]
    diff = sj - s[:, None, :]
    w = jnp.exp(-10.0 * jnp.sum(diff * diff, axis=-1))
    msg = hj * w[:, :, None]
    agg = jnp.concatenate([jnp.mean(msg, axis=1), jnp.max(msg, axis=1)],
                          axis=-1)
    return x @ p["lin_out1"]["W"] + _linear(agg, p["lin_out2"])


def _gravnet_block(x, batch, batchf, lims, p, ns):
    x = _gravnet_conv(x, batch, batchf, lims, p["conv"])
    x = _batchnorm(x, p["bn1"])
    x = jnp.tanh(_linear(x, p["lin1"]))
    x = _batchnorm(x, p["bn2"])
    x = jnp.tanh(_linear(x, p["lin2"]))
    x = _global_exchange(x, batch, ns)
    x = jnp.tanh(_linear(x, p["out_lin"]))
    x = _batchnorm(x, p["out_bn"])
    return x


def kernel(x, batch, params):
    batchf = batch.astype(jnp.float32)
    starts = jnp.searchsorted(batch, jnp.arange(NUM_EVENTS, dtype=jnp.int32),
                              side="left").astype(jnp.int32)
    ends = jnp.searchsorted(batch, jnp.arange(NUM_EVENTS, dtype=jnp.int32),
                            side="right").astype(jnp.int32)
    b_first = batch[::RB]
    b_last = batch[RB - 1::RB]
    lo = (starts[b_first] // CT).astype(jnp.int32)
    hi = ((ends[b_last] + CT - 1) // CT).astype(jnp.int32)
    lims = jnp.stack([lo, hi], axis=1)

    x = _batchnorm(x, params["bn0"])
    x = _global_exchange(x, batch, NUM_EVENTS)
    x = _linear(x, params["input"])
    outs = []
    for bp in params["blocks"]:
        x = _gravnet_block(x, batch, batchf, lims, bp, NUM_EVENTS)
        outs.append(x)
    x = jnp.concatenate(outs, axis=-1)
    for dp in params["dense"]:
        x = _batchnorm(jax.nn.relu(_linear(x, dp["lin"])), dp["bn"])
    x = jax.nn.relu(_linear(x, params["out1"]))
    x = jax.nn.relu(_linear(x, params["out2"]))
    return _linear(x, params["out3"])
